# baseline (speedup 1.0000x reference)
import jax
import jax.numpy as jnp
from jax.experimental import pallas as pl

_H1, _H2 = 64, 32


def _msgcn_body(x1_ref, a1_ref, x2_ref, a2_ref,
                w1_ref, b1_ref, w2_ref, b2_ref, out_ref):
    w1 = w1_ref[...]
    b1 = b1_ref[...]
    w2 = w2_ref[...]
    b2 = b2_ref[...]

    def branch(x, a):
        h = jnp.dot(x, w1, preferred_element_type=jnp.float32)
        h = jnp.dot(a, h, preferred_element_type=jnp.float32) + b1
        h = jnp.maximum(h, 0.0)
        h = jnp.dot(h, w2, preferred_element_type=jnp.float32)
        h = jnp.dot(a, h, preferred_element_type=jnp.float32) + b2
        return jnp.maximum(h, 0.0)

    o1 = branch(x1_ref[0], a1_ref[0])
    o2 = branch(x2_ref[0], a2_ref[0])
    d = o1 - o2
    d2 = jnp.sum(d * d, axis=(0, 1), keepdims=True)
    out_ref[0] = jnp.sqrt(d2 + 1e-12)


def _scale_distance(x1, x2, a1, a2, W1, b1, W2, b2):
    bsz, n, _ = x1.shape
    out = pl.pallas_call(
        _msgcn_body,
        grid=(bsz,),
        in_specs=[
            pl.BlockSpec((1, n, n), lambda i: (i, 0, 0)),
            pl.BlockSpec((1, n, n), lambda i: (i, 0, 0)),
            pl.BlockSpec((1, n, n), lambda i: (i, 0, 0)),
            pl.BlockSpec((1, n, n), lambda i: (i, 0, 0)),
            pl.BlockSpec((n, _H1), lambda i: (0, 0)),
            pl.BlockSpec((1, _H1), lambda i: (0, 0)),
            pl.BlockSpec((_H1, _H2), lambda i: (0, 0)),
            pl.BlockSpec((1, _H2), lambda i: (0, 0)),
        ],
        out_specs=pl.BlockSpec((1, 1, 1), lambda i: (i, 0, 0)),
        out_shape=jax.ShapeDtypeStruct((bsz, 1, 1), jnp.float32),
    )(x1, a1, x2, a2, W1, b1.reshape(1, _H1), W2, b2.reshape(1, _H2))
    return out[:, 0, 0]


def kernel(sub1a, sub2a, adj1a, adj2a, W1a, b1a, W2a, b2a,
           sub1b, sub2b, adj1b, adj2b, W1b, b1b, W2b, b2b,
           sub1c, sub2c, adj1c, adj2c, W1c, b1c, W2c, b2c,
           sub1d, sub2d, adj1d, adj2d, W1d, b1d, W2d, b2d):
    return (
        _scale_distance(sub1a, sub2a, adj1a, adj2a, W1a, b1a, W2a, b2a),
        _scale_distance(sub1b, sub2b, adj1b, adj2b, W1b, b1b, W2b, b2b),
        _scale_distance(sub1c, sub2c, adj1c, adj2c, W1c, b1c, W2c, b2c),
        _scale_distance(sub1d, sub2d, adj1d, adj2d, W1d, b1d, W2d, b2d),
    )


# speedup vs baseline: 1.1999x; 1.1999x over previous
import functools

import jax
import jax.numpy as jnp
from jax.experimental import pallas as pl

_H1, _H2 = 64, 32


def _msgcn_body(x1_ref, a1_ref, x2_ref, a2_ref,
                w1_ref, b1_ref, w2_ref, b2_ref, out_ref, *, bb):
    w1 = w1_ref[...]
    b1 = b1_ref[...]
    w2 = w2_ref[...]
    b2 = b2_ref[...]

    def branch(x, a):
        h = jnp.dot(x, w1, preferred_element_type=jnp.float32)
        h = jnp.dot(a, h, preferred_element_type=jnp.float32) + b1
        h = jnp.maximum(h, 0.0)
        h = jnp.dot(h, w2, preferred_element_type=jnp.float32)
        h = jnp.dot(a, h, preferred_element_type=jnp.float32) + b2
        return jnp.maximum(h, 0.0)

    for j in range(bb):
        o1 = branch(x1_ref[j], a1_ref[j])
        o2 = branch(x2_ref[j], a2_ref[j])
        d = o1 - o2
        d2 = jnp.sum(d * d, axis=(0, 1), keepdims=True)
        out_ref[j] = jnp.sqrt(d2 + 1e-12)


def _scale_distance(x1, x2, a1, a2, W1, b1, W2, b2, bb=8):
    bsz, n, _ = x1.shape
    grid = bsz // bb
    out = pl.pallas_call(
        functools.partial(_msgcn_body, bb=bb),
        grid=(grid,),
        in_specs=[
            pl.BlockSpec((bb, n, n), lambda i: (i, 0, 0)),
            pl.BlockSpec((bb, n, n), lambda i: (i, 0, 0)),
            pl.BlockSpec((bb, n, n), lambda i: (i, 0, 0)),
            pl.BlockSpec((bb, n, n), lambda i: (i, 0, 0)),
            pl.BlockSpec((n, _H1), lambda i: (0, 0)),
            pl.BlockSpec((1, _H1), lambda i: (0, 0)),
            pl.BlockSpec((_H1, _H2), lambda i: (0, 0)),
            pl.BlockSpec((1, _H2), lambda i: (0, 0)),
        ],
        out_specs=pl.BlockSpec((bb, 1, 1), lambda i: (i, 0, 0)),
        out_shape=jax.ShapeDtypeStruct((bsz, 1, 1), jnp.float32),
    )(x1, a1, x2, a2, W1, b1.reshape(1, _H1), W2, b2.reshape(1, _H2))
    return out[:, 0, 0]


def kernel(sub1a, sub2a, adj1a, adj2a, W1a, b1a, W2a, b2a,
           sub1b, sub2b, adj1b, adj2b, W1b, b1b, W2b, b2b,
           sub1c, sub2c, adj1c, adj2c, W1c, b1c, W2c, b2c,
           sub1d, sub2d, adj1d, adj2d, W1d, b1d, W2d, b2d):
    return (
        _scale_distance(sub1a, sub2a, adj1a, adj2a, W1a, b1a, W2a, b2a),
        _scale_distance(sub1b, sub2b, adj1b, adj2b, W1b, b1b, W2b, b2b),
        _scale_distance(sub1c, sub2c, adj1c, adj2c, W1c, b1c, W2c, b2c),
        _scale_distance(sub1d, sub2d, adj1d, adj2d, W1d, b1d, W2d, b2d),
    )
